# baseline (speedup 1.0000x reference)
import functools

import jax
import jax.numpy as jnp
from jax import lax
from jax.experimental import pallas as pl
from jax.experimental.pallas import tpu as pltpu
from jax.experimental.pallas import tpu_sc as plsc

VOCAB = 100000
EMBED_DIM = 128
OUT_DIM = 2
BATCH = 4096
SEQ = 200

NC, NS, L = 2, 16, 16
VB = 20480
VPAD = ((VOCAB + VB - 1) // VB) * VB
ROWS_PER_G = BATCH // NS
NJ = ROWS_PER_G // L
ST = SEQ // 8
RT = BATCH // 128


def _proj_body(w_ref, b_ref, emb_ref, out0_ref, out1_ref):
    res = lax.dot_general(
        w_ref[...], emb_ref[...],
        (((1,), (1,)), ((), ())),
        preferred_element_type=jnp.float32,
    ) * (1.0 / SEQ)
    out0_ref[...] = (res[0] + b_ref[0] * (1.0 / SEQ)).astype(jnp.bfloat16)
    out1_ref[...] = (res[1] + b_ref[1] * (1.0 / SEQ)).astype(jnp.bfloat16)


def _project_table(W, b, emb_table):
    return pl.pallas_call(
        _proj_body,
        grid=(VPAD // VB,),
        in_specs=[
            pl.BlockSpec((OUT_DIM, EMBED_DIM), lambda i: (0, 0)),
            pl.BlockSpec(memory_space=pltpu.SMEM),
            pl.BlockSpec((VB, EMBED_DIM), lambda i: (i, 0)),
        ],
        out_specs=[
            pl.BlockSpec((VB,), lambda i: (i,)),
            pl.BlockSpec((VB,), lambda i: (i,)),
        ],
        out_shape=[
            jax.ShapeDtypeStruct((VPAD,), jnp.bfloat16),
            jax.ShapeDtypeStruct((VPAD,), jnp.bfloat16),
        ],
    )(W, b, emb_table)


def _sc_pool_body(proj0_hbm, proj1_hbm, tok_hbm, out_hbm,
                  table_v, tokbuf_v, out_v, sem_t, sem_c):
    comp = lax.axis_index("c")
    g = lax.axis_index("s")

    cp_tok = pltpu.async_copy(tok_hbm.at[:, pl.ds(2 * g, 2)], tokbuf_v, sem_c)

    @pl.when(comp == 0)
    def _():
        pltpu.async_copy(proj0_hbm, table_v, sem_t)

    @pl.when(comp == 1)
    def _():
        pltpu.async_copy(proj1_hbm, table_v, sem_t)

    pltpu.make_async_copy(proj0_hbm, table_v, sem_t).wait()
    cp_tok.wait()

    himask = jnp.int32(-65536)

    def body(st, accs):
        new = list(accs)
        for si in range(8):
            for j in range(NJ):
                idx = tokbuf_v[st, j // 8, si, pl.ds((j % 8) * L, L)]
                w = plsc.load_gather(table_v, [lax.shift_right_logical(idx, 1)])
                lo_f = plsc.bitcast(lax.shift_left(w, 16), jnp.float32)
                hi_f = plsc.bitcast(lax.bitwise_and(w, himask), jnp.float32)
                odd = lax.eq(lax.bitwise_and(idx, 1), 1)
                new[j] = new[j] + lax.select(odd, hi_f, lo_f)
        return tuple(new)

    accs = lax.fori_loop(
        0, ST, body, tuple(jnp.zeros((L,), jnp.float32) for _ in range(NJ))
    )
    for j in range(NJ):
        out_v[pl.ds(j * L, L)] = accs[j]
    pltpu.sync_copy(out_v, out_hbm.at[comp, pl.ds(g * ROWS_PER_G, ROWS_PER_G)])


_sc_pool = functools.partial(
    pl.kernel,
    out_type=jax.ShapeDtypeStruct((OUT_DIM, BATCH), jnp.float32),
    mesh=plsc.VectorSubcoreMesh(core_axis_name="c", subcore_axis_name="s"),
    compiler_params=pltpu.CompilerParams(needs_layout_passes=False),
    scratch_types=[
        pltpu.VMEM((VPAD // 2,), jnp.int32),
        pltpu.VMEM((ST, 2, 8, 128), jnp.int32),
        pltpu.VMEM((ROWS_PER_G,), jnp.float32),
        pltpu.SemaphoreType.DMA,
        pltpu.SemaphoreType.DMA,
    ],
)(_sc_pool_body)


def kernel(text_token, emb_table, W, b):
    tok = text_token.astype(jnp.int32)
    proj0_bf, proj1_bf = _project_table(W, b, emb_table)
    proj0 = lax.bitcast_convert_type(
        proj0_bf.reshape(VPAD // 2, 2), jnp.int32)
    proj1 = lax.bitcast_convert_type(
        proj1_bf.reshape(VPAD // 2, 2), jnp.int32)
    tok4 = tok.T.reshape(ST, 8, RT, 128).transpose(0, 2, 1, 3)
    return _sc_pool(proj0, proj1, tok4).T


# speedup vs baseline: 2.5145x; 2.5145x over previous
import functools

import jax
import jax.numpy as jnp
from jax import lax
from jax.experimental import pallas as pl
from jax.experimental.pallas import tpu as pltpu
from jax.experimental.pallas import tpu_sc as plsc

VOCAB = 100000
EMBED_DIM = 128
OUT_DIM = 2
BATCH = 4096
SEQ = 200

NC, NS, L = 2, 16, 16
VB = 20480
VPAD = ((VOCAB + VB - 1) // VB) * VB
ROWS_PER_G = BATCH // NS
NJ = ROWS_PER_G // L
ST = SEQ // 8
RT = BATCH // 128


def _proj_body(w_ref, b_ref, emb_ref, out0_ref, out1_ref):
    res = lax.dot_general(
        w_ref[...], emb_ref[...],
        (((1,), (1,)), ((), ())),
        preferred_element_type=jnp.float32,
    ) * (1.0 / SEQ)
    out0_ref[...] = res[0] + b_ref[0] * (1.0 / SEQ)
    out1_ref[...] = res[1] + b_ref[1] * (1.0 / SEQ)


def _project_table(W, b, emb_table):
    return pl.pallas_call(
        _proj_body,
        grid=(VPAD // VB,),
        in_specs=[
            pl.BlockSpec((OUT_DIM, EMBED_DIM), lambda i: (0, 0)),
            pl.BlockSpec(memory_space=pltpu.SMEM),
            pl.BlockSpec((VB, EMBED_DIM), lambda i: (i, 0)),
        ],
        out_specs=[
            pl.BlockSpec((VB,), lambda i: (i,)),
            pl.BlockSpec((VB,), lambda i: (i,)),
        ],
        out_shape=[
            jax.ShapeDtypeStruct((VPAD,), jnp.float32),
            jax.ShapeDtypeStruct((VPAD,), jnp.float32),
        ],
    )(W, b, emb_table)


def _sc_pool_body(proj0_hbm, proj1_hbm, tok_hbm, out_hbm,
                  table_v, tokbuf_v, out_v, sem_t, sem_c):
    comp = lax.axis_index("c")
    g = lax.axis_index("s")

    cp_tok = pltpu.async_copy(tok_hbm.at[:, pl.ds(2 * g, 1)], tokbuf_v, sem_c)

    @pl.when(comp == 0)
    def _():
        pltpu.async_copy(proj0_hbm, table_v, sem_t)

    @pl.when(comp == 1)
    def _():
        pltpu.async_copy(proj1_hbm, table_v, sem_t)

    pltpu.make_async_copy(proj0_hbm, table_v, sem_t).wait()
    cp_tok.wait()

    for k in range(2):
        def body(st, accs):
            new = list(accs)
            for si in range(8):
                for jj in range(8):
                    idx = tokbuf_v[st, 0, si, pl.ds(jj * L, L)]
                    new[jj] = new[jj] + plsc.load_gather(table_v, [idx])
            return tuple(new)

        accs = lax.fori_loop(
            0, ST, body, tuple(jnp.zeros((L,), jnp.float32) for _ in range(8))
        )
        for jj in range(8):
            out_v[pl.ds((k * 8 + jj) * L, L)] = accs[jj]
        if k == 0:
            pltpu.sync_copy(tok_hbm.at[:, pl.ds(2 * g + 1, 1)], tokbuf_v)

    pltpu.sync_copy(out_v, out_hbm.at[comp, pl.ds(g * ROWS_PER_G, ROWS_PER_G)])


_sc_pool = functools.partial(
    pl.kernel,
    out_type=jax.ShapeDtypeStruct((OUT_DIM, BATCH), jnp.float32),
    mesh=plsc.VectorSubcoreMesh(core_axis_name="c", subcore_axis_name="s"),
    compiler_params=pltpu.CompilerParams(needs_layout_passes=False),
    scratch_types=[
        pltpu.VMEM((VPAD,), jnp.float32),
        pltpu.VMEM((ST, 1, 8, 128), jnp.int32),
        pltpu.VMEM((ROWS_PER_G,), jnp.float32),
        pltpu.SemaphoreType.DMA,
        pltpu.SemaphoreType.DMA,
    ],
)(_sc_pool_body)


def kernel(text_token, emb_table, W, b):
    tok = text_token.astype(jnp.int32)
    proj0, proj1 = _project_table(W, b, emb_table)
    tok4 = tok.T.reshape(ST, 8, RT, 128).transpose(0, 2, 1, 3)
    return _sc_pool(proj0, proj1, tok4).T
